# baseline (speedup 1.0000x reference)
import functools

import jax
import jax.numpy as jnp
from jax import lax
from jax.experimental import pallas as pl
from jax.experimental.pallas import tpu as pltpu
from jax.experimental.pallas import tpu_sc as plsc

VOC = 100000
EMB = 64
PAD = 128
SCALE = 8.0

NC = 2
NS = 16
NW = NC * NS

NB = 4096
NT = 200
NSPLIT = 2
TOK = NB * NT // NSPLIT
TPW = TOK // NW
CHUNK = 128
NCH = TPW // CHUNK
NBUF = 2
UNROLL = 8

_mesh = plsc.VectorSubcoreMesh(core_axis_name="c", subcore_axis_name="s")
_params = pltpu.CompilerParams(use_tc_tiling_on_sc=True)


@functools.partial(
    pl.kernel,
    mesh=_mesh,
    out_type=jax.ShapeDtypeStruct((TOK, EMB), jnp.float32),
    scratch_types=[
        pltpu.VMEM((TPW,), jnp.int32),
        pltpu.VMEM((NBUF, CHUNK, PAD), jnp.float32),
        pltpu.VMEM((NBUF, CHUNK, EMB), jnp.float32),
        [pltpu.SemaphoreType.DMA] * NBUF,
        [pltpu.SemaphoreType.DMA] * NBUF,
    ],
    compiler_params=_params,
)
def _emb_lookup(tokens_hbm, table_hbm, out_hbm, idx_v, rows_v, comp_v,
                gsems, ssems):
    wid = lax.axis_index("s") * NC + lax.axis_index("c")
    base = wid * TPW

    pltpu.sync_copy(tokens_hbm.at[pl.ds(base, TPW)], idx_v)

    def start_gather(c, b):
        pltpu.async_copy(
            table_hbm.at[idx_v.at[pl.ds(c * CHUNK, CHUNK)]],
            rows_v.at[b],
            gsems[b],
        )

    def wait_gather(c, b):
        pltpu.make_async_copy(
            table_hbm.at[idx_v.at[pl.ds(c * CHUNK, CHUNK)]],
            rows_v.at[b],
            gsems[b],
        ).wait()

    def scale_compact(b):
        def body(g, _):
            r0 = g * UNROLL
            for k in range(UNROLL):
                for j in range(EMB // 16):
                    comp_v[b, r0 + k, pl.ds(j * 16, 16)] = (
                        rows_v[b, r0 + k, pl.ds(j * 16, 16)] * SCALE
                    )
            return 0

        lax.fori_loop(0, CHUNK // UNROLL, body, 0)

    def start_write(c, b):
        pltpu.async_copy(
            comp_v.at[b], out_hbm.at[pl.ds(base + c * CHUNK, CHUNK)], ssems[b]
        )

    def wait_write(c, b):
        pltpu.make_async_copy(
            comp_v.at[b], out_hbm.at[pl.ds(base + c * CHUNK, CHUNK)], ssems[b]
        ).wait()

    for b in range(NBUF):
        start_gather(b, b)

    for b in range(NBUF):
        wait_gather(b, b)
        scale_compact(b)
        start_gather(b + NBUF, b)
        start_write(b, b)

    def outer(i, _):
        for b in range(NBUF):
            c = i * NBUF + b
            wait_gather(c, b)
            wait_write(c - NBUF, b)
            scale_compact(b)
            start_gather(c + NBUF, b)
            start_write(c, b)
        return 0

    lax.fori_loop(1, NCH // NBUF - 1, outer, 0)

    for b in range(NBUF):
        c = (NCH // NBUF - 1) * NBUF + b
        wait_gather(c, b)
        wait_write(c - NBUF, b)
        scale_compact(b)
        start_write(c, b)

    for b in range(NBUF):
        wait_write((NCH // NBUF - 1) * NBUF + b, b)


def kernel(tokens, table):
    table128 = jnp.pad(table, ((0, 0), (0, PAD - EMB)))
    flat = tokens.reshape(-1)
    halves = [
        _emb_lookup(lax.slice(flat, (s * TOK,), ((s + 1) * TOK,)), table128)
        for s in range(NSPLIT)
    ]
    return jnp.concatenate(halves, axis=0).reshape(NB, NT, EMB)


# speedup vs baseline: 1.4109x; 1.4109x over previous
import functools

import jax
import jax.numpy as jnp
from jax import lax
from jax.experimental import pallas as pl
from jax.experimental.pallas import tpu as pltpu
from jax.experimental.pallas import tpu_sc as plsc

VOC = 100000
EMB = 64
PAD = 128
SCALE = 8.0

NC = 2
NS = 16
NW = NC * NS

NB = 4096
NT = 200
TPW = NB * NT // NW
CHUNK = 128
NCH = TPW // CHUNK
NBUF = 2
UNROLL = 8

_mesh = plsc.VectorSubcoreMesh(core_axis_name="c", subcore_axis_name="s")
_params = pltpu.CompilerParams(use_tc_tiling_on_sc=True)


@functools.partial(
    pl.kernel,
    mesh=_mesh,
    out_type=jax.ShapeDtypeStruct((NB * NT, EMB), jnp.float32),
    scratch_types=[
        pltpu.VMEM((TPW,), jnp.int32),
        pltpu.VMEM((NBUF, CHUNK, PAD), jnp.float32),
        pltpu.VMEM((NBUF, CHUNK, EMB), jnp.float32),
        [pltpu.SemaphoreType.DMA] * NBUF,
        [pltpu.SemaphoreType.DMA] * NBUF,
    ],
    compiler_params=_params,
)
def _emb_lookup(tokens_hbm, table_hbm, out_hbm, idx_v, rows_v, comp_v,
                gsems, ssems):
    wid = lax.axis_index("s") * NC + lax.axis_index("c")
    base = wid * TPW

    pltpu.sync_copy(tokens_hbm.at[pl.ds(base, TPW)], idx_v)

    def start_gather(c, b):
        pltpu.async_copy(
            table_hbm.at[idx_v.at[pl.ds(c * CHUNK, CHUNK)]],
            rows_v.at[b],
            gsems[b],
        )

    def wait_gather(c, b):
        pltpu.make_async_copy(
            table_hbm.at[idx_v.at[pl.ds(c * CHUNK, CHUNK)]],
            rows_v.at[b],
            gsems[b],
        ).wait()

    def compact(b):
        def body(g, _):
            r0 = g * UNROLL
            for k in range(UNROLL):
                for j in range(EMB // 16):
                    comp_v[b, r0 + k, pl.ds(j * 16, 16)] = (
                        rows_v[b, r0 + k, pl.ds(j * 16, 16)]
                    )
            return 0

        lax.fori_loop(0, CHUNK // UNROLL, body, 0)

    def start_write(c, b):
        pltpu.async_copy(
            comp_v.at[b], out_hbm.at[pl.ds(base + c * CHUNK, CHUNK)], ssems[b]
        )

    def wait_write(c, b):
        pltpu.make_async_copy(
            comp_v.at[b], out_hbm.at[pl.ds(base + c * CHUNK, CHUNK)], ssems[b]
        ).wait()

    for b in range(NBUF):
        start_gather(b, b)

    for b in range(NBUF):
        wait_gather(b, b)
        compact(b)
        start_gather(b + NBUF, b)
        start_write(b, b)

    def outer(i, _):
        for b in range(NBUF):
            c = i * NBUF + b
            wait_gather(c, b)
            wait_write(c - NBUF, b)
            compact(b)
            start_gather(c + NBUF, b)
            start_write(c, b)
        return 0

    lax.fori_loop(1, NCH // NBUF - 1, outer, 0)

    for b in range(NBUF):
        c = (NCH // NBUF - 1) * NBUF + b
        wait_gather(c, b)
        wait_write(c - NBUF, b)
        compact(b)
        start_write(c, b)

    for b in range(NBUF):
        wait_write((NCH // NBUF - 1) * NBUF + b, b)


def kernel(tokens, table):
    table128 = jnp.pad(table * SCALE, ((0, 0), (0, PAD - EMB)))
    out = _emb_lookup(tokens.reshape(-1), table128)
    return out.reshape(NB, NT, EMB)


# speedup vs baseline: 1.4987x; 1.0623x over previous
import functools

import jax
import jax.numpy as jnp
from jax import lax
from jax.experimental import pallas as pl
from jax.experimental.pallas import tpu as pltpu
from jax.experimental.pallas import tpu_sc as plsc

VOC = 100000
EMB = 64
PAD = 128
SCALE = 8.0

NC = 2
NS = 16
NW = NC * NS

NB = 4096
NT = 200
TPW = NB * NT // NW
CHUNK = 128
NCH = TPW // CHUNK
NBUF = 2
UNROLL = 8

_mesh = plsc.VectorSubcoreMesh(core_axis_name="c", subcore_axis_name="s")
_params = pltpu.CompilerParams(use_tc_tiling_on_sc=True)


@functools.partial(
    pl.kernel,
    mesh=_mesh,
    out_type=jax.ShapeDtypeStruct((NB * NT, EMB), jnp.float32),
    scratch_types=[
        pltpu.VMEM((TPW,), jnp.int32),
        pltpu.VMEM((NBUF, CHUNK, PAD), jnp.float32),
        pltpu.VMEM((NBUF, CHUNK, EMB), jnp.float32),
        [pltpu.SemaphoreType.DMA] * NBUF,
        [pltpu.SemaphoreType.DMA] * NBUF,
    ],
    compiler_params=_params,
)
def _emb_lookup(tokens_hbm, table_hbm, out_hbm, idx_v, rows_v, comp_v,
                gsems, ssems):
    wid = lax.axis_index("s") * NC + lax.axis_index("c")
    base = wid * TPW

    pltpu.sync_copy(tokens_hbm.at[pl.ds(base, TPW)], idx_v)

    def start_gather(c, b):
        pltpu.async_copy(
            table_hbm.at[idx_v.at[pl.ds(c * CHUNK, CHUNK)]],
            rows_v.at[b],
            gsems[b],
        )

    def wait_gather(c, b):
        pltpu.make_async_copy(
            table_hbm.at[idx_v.at[pl.ds(c * CHUNK, CHUNK)]],
            rows_v.at[b],
            gsems[b],
        ).wait()

    def scale_compact(b):
        def body(g, _):
            r0 = g * UNROLL
            for k in range(UNROLL):
                for j in range(EMB // 16):
                    comp_v[b, r0 + k, pl.ds(j * 16, 16)] = (
                        rows_v[b, r0 + k, pl.ds(j * 16, 16)] * SCALE
                    )
            return 0

        lax.fori_loop(0, CHUNK // UNROLL, body, 0)

    def start_write(c, b):
        pltpu.async_copy(
            comp_v.at[b], out_hbm.at[pl.ds(base + c * CHUNK, CHUNK)], ssems[b]
        )

    def wait_write(c, b):
        pltpu.make_async_copy(
            comp_v.at[b], out_hbm.at[pl.ds(base + c * CHUNK, CHUNK)], ssems[b]
        ).wait()

    for b in range(NBUF):
        start_gather(b, b)

    for b in range(NBUF):
        wait_gather(b, b)
        scale_compact(b)
        start_gather(b + NBUF, b)
        start_write(b, b)

    def outer(i, _):
        for b in range(NBUF):
            c = i * NBUF + b
            wait_gather(c, b)
            wait_write(c - NBUF, b)
            scale_compact(b)
            start_gather(c + NBUF, b)
            start_write(c, b)
        return 0

    lax.fori_loop(1, NCH // NBUF - 1, outer, 0)

    for b in range(NBUF):
        c = (NCH // NBUF - 1) * NBUF + b
        wait_gather(c, b)
        wait_write(c - NBUF, b)
        scale_compact(b)
        start_write(c, b)

    for b in range(NBUF):
        wait_write((NCH // NBUF - 1) * NBUF + b, b)


def kernel(tokens, table):
    table128 = jnp.pad(table, ((0, 0), (0, PAD - EMB)))
    out = _emb_lookup(tokens.reshape(-1), table128)
    return out.reshape(NB, NT, EMB)
